# baseline (speedup 1.0000x reference)
import functools

import jax
import jax.numpy as jnp
from jax import lax
from jax.experimental import pallas as pl
from jax.experimental.pallas import tpu as pltpu
from jax.experimental.pallas import tpu_sc as plsc

MIN_NORM = 1e-15
MAXNORM = 1.0 - 4e-3

DP = 48
CHUNK = 128
NW = 32


def _artanh(x):
    x = jnp.clip(x, -1.0 + 1e-7, 1.0 - 1e-7)
    return 0.5 * jnp.log((1.0 + x) / (1.0 - x))


def _rownorm(x):
    return jnp.clip(jnp.sqrt(jnp.sum(x * x, axis=-1, keepdims=True)), MIN_NORM)


def _projf(x):
    norm = _rownorm(x)
    return jnp.where(norm > MAXNORM, x / norm * MAXNORM, x)


def _expmap0f(u):
    un = _rownorm(u)
    return jnp.tanh(un) * u / un


def _logmap0f(p):
    pn = _rownorm(p)
    return _artanh(pn) * p / pn


def _pre_body(x_ref, w_ref, b_ref, out_ref):
    x = x_ref[...]
    w = w_ref[...]
    b = b_ref[...]
    x_norm = _rownorm(x)
    mx = lax.dot_general(x, w, (((1,), (1,)), ((), ())),
                         preferred_element_type=jnp.float32)
    mx_norm = _rownorm(mx)
    res_c = jnp.tanh(mx_norm / x_norm * _artanh(x_norm)) * mx / mx_norm
    cond = jnp.all(mx == 0.0, axis=-1, keepdims=True)
    res = jnp.where(cond, jnp.zeros_like(res_c), res_c)
    res = _projf(res)
    hb = _projf(_expmap0f(b))
    x2 = jnp.sum(res * res, axis=-1, keepdims=True)
    y2 = jnp.sum(hb * hb, axis=-1, keepdims=True)
    xy = jnp.sum(res * hb, axis=-1, keepdims=True)
    num = (1.0 + 2.0 * xy + y2) * res + (1.0 - x2) * hb
    den = jnp.clip(1.0 + 2.0 * xy + x2 * y2, MIN_NORM)
    res = _projf(num / den)
    out_ref[...] = _logmap0f(res)


def _post_body(s0_ref, s1_ref, out_ref):
    s = s0_ref[0] + s1_ref[0]
    h = _projf(_expmap0f(s))
    h = _projf(_expmap0f(_logmap0f(h)))
    out_ref[...] = _logmap0f(h)[:, :out_ref.shape[-1]]


def _sc_agg(xt, src, dst, ew, n_pad, n0, n1):
    rows_per_tile = n_pad // 16
    n_zero_copies = rows_per_tile // CHUNK
    epw0, epw1 = n0 * CHUNK, n1 * CHUNK
    mesh = plsc.VectorSubcoreMesh(core_axis_name="c", subcore_axis_name="s")

    @functools.partial(
        pl.kernel,
        out_type=jax.ShapeDtypeStruct((2, n_pad, DP), jnp.float32),
        mesh=mesh,
        compiler_params=pltpu.CompilerParams(use_tc_tiling_on_sc=False),
        scratch_types=(
            [pltpu.VMEM((CHUNK,), jnp.int32) for _ in range(3)]
            + [pltpu.VMEM((CHUNK,), jnp.int32) for _ in range(3)]
            + [pltpu.VMEM((CHUNK,), jnp.float32) for _ in range(3)]
            + [pltpu.VMEM((CHUNK, DP), jnp.float32) for _ in range(3)]
            + [pltpu.VMEM_SHARED((n_pad, DP), jnp.float32)]
            + [pltpu.SemaphoreType.DMA for _ in range(9)]
        ),
    )
    def body(xt_hbm, src_hbm, dst_hbm, w_hbm, out_hbm,
             s0, s1, s2, d0, d1, d2, w0, w1, w2, r0, r1, r2,
             acc_sh, gs0, gs1, gs2, is0, is1, is2, ss0, ss1, ss2):
        src_v = [s0, s1, s2]
        dst_v = [d0, d1, d2]
        w_v = [w0, w1, w2]
        rows_v = [r0, r1, r2]
        gsem = [gs0, gs1, gs2]
        isem = [is0, is1, is2]
        ssem = [ss0, ss1, ss2]

        cid = lax.axis_index("c")
        sid = lax.axis_index("s")
        base = jnp.where(cid == 0, sid * epw0, 16 * epw0 + sid * epw1)
        nc = jnp.where(cid == 0, n0, n1)
        zero = jnp.zeros((16,), jnp.float32)

        def zrow(i, carry):
            for j in range(DP // 16):
                r0[i, pl.ds(j * 16, 16)] = zero
            return carry
        lax.fori_loop(0, CHUNK, zrow, 0)
        for t in range(n_zero_copies):
            pltpu.sync_copy(
                r0, acc_sh.at[pl.ds(sid * rows_per_tile + t * CHUNK, CHUNK)])
        plsc.subcore_barrier()

        def issue_idx(ci, p):
            off = base + ci * CHUNK
            pltpu.async_copy(src_hbm.at[pl.ds(off, CHUNK)], src_v[p], isem[p])
            pltpu.async_copy(dst_hbm.at[pl.ds(off, CHUNK)], dst_v[p], isem[p])
            pltpu.async_copy(w_hbm.at[pl.ds(off, CHUNK)], w_v[p], isem[p])

        def wait_idx(p):
            pltpu.make_async_copy(
                src_hbm.at[pl.ds(0, CHUNK)], src_v[p], isem[p]).wait()
            pltpu.make_async_copy(
                dst_hbm.at[pl.ds(0, CHUNK)], dst_v[p], isem[p]).wait()
            pltpu.make_async_copy(
                w_hbm.at[pl.ds(0, CHUNK)], w_v[p], isem[p]).wait()

        issue_idx(0, 0)
        issue_idx(1, 1)
        wait_idx(0)
        pltpu.async_copy(xt_hbm.at[src_v[0]], rows_v[0], gsem[0])

        def phase(co, ph):
            p0 = ph % 3
            p1 = (ph + 1) % 3
            p2 = (ph + 2) % 3
            ci = co * 3 + ph
            pltpu.make_async_copy(
                xt_hbm.at[src_v[p0]], rows_v[p0], gsem[p0]).wait()

            def escale(g, c2):
                wv = w_v[p0][pl.ds(g * 16, 16)]
                for i in range(16):
                    ws = jnp.full((16,), wv[i], jnp.float32)
                    e = g * 16 + i
                    for j in range(DP // 16):
                        rows_v[p0][e, pl.ds(j * 16, 16)] = (
                            rows_v[p0][e, pl.ds(j * 16, 16)] * ws)
                return c2
            lax.fori_loop(0, CHUNK // 16, escale, 0)

            pltpu.async_copy(
                rows_v[p0], acc_sh.at[dst_v[p0]], ssem[p0], add=True)

            @pl.when(ci > 0)
            def _():
                pltpu.make_async_copy(
                    rows_v[p2], acc_sh.at[dst_v[p2]], ssem[p2]).wait()
            issue_idx(ci + 2, p2)
            wait_idx(p1)
            pltpu.async_copy(xt_hbm.at[src_v[p1]], rows_v[p1], gsem[p1])

        def cycle(co, carry):
            for ph in range(3):
                phase(co, ph)
            return carry
        lax.fori_loop(0, nc // 3, cycle, 0)

        pltpu.make_async_copy(
            xt_hbm.at[src_v[0]], rows_v[0], gsem[0]).wait()
        wait_idx(1)
        pltpu.make_async_copy(
            rows_v[2], acc_sh.at[dst_v[2]], ssem[2]).wait()

        plsc.subcore_barrier()
        pltpu.sync_copy(acc_sh.at[pl.ds(sid * rows_per_tile, rows_per_tile)],
                        out_hbm.at[cid, pl.ds(sid * rows_per_tile, rows_per_tile)])

    return body(xt, src, dst, ew)


def kernel(x, edge_index, edge_weight, weight, bias):
    n, d = x.shape
    k = weight.shape[0]
    e = edge_index.shape[1]

    wp = jnp.zeros((DP, d), jnp.float32).at[:k].set(weight)
    bp = jnp.zeros((1, DP), jnp.float32).at[0, :k].set(bias)

    rows = 2000
    grid = n // rows
    xt = pl.pallas_call(
        _pre_body,
        grid=(grid,),
        in_specs=[
            pl.BlockSpec((rows, d), lambda i: (i, 0)),
            pl.BlockSpec((DP, d), lambda i: (0, 0)),
            pl.BlockSpec((1, DP), lambda i: (0, 0)),
        ],
        out_specs=pl.BlockSpec((rows, DP), lambda i: (i, 0)),
        out_shape=jax.ShapeDtypeStruct((n, DP), jnp.float32),
    )(x, wp, bp)

    tot = 9 * (-(-e // (16 * CHUNK * 9)))
    n1 = 3 * (-(-(tot * 5) // (18 * 3)))
    n0 = tot - n1
    e_pad = 16 * tot * CHUNK
    e_alloc = e_pad + 2 * CHUNK
    n_pad = 16 * (-(-(n + 1) // (16 * CHUNK)) * CHUNK)
    src = jnp.pad(edge_index[0], (0, e_alloc - e))
    dst = jnp.pad(edge_index[1], (0, e_alloc - e), constant_values=n)
    ew = jnp.pad(edge_weight, (0, e_alloc - e))
    parts = _sc_agg(xt, src, dst, ew, n_pad, n0, n1)

    out = pl.pallas_call(
        _post_body,
        grid=(grid,),
        in_specs=[
            pl.BlockSpec((1, rows, DP), lambda i: (0, i, 0)),
            pl.BlockSpec((1, rows, DP), lambda i: (1, i, 0)),
        ],
        out_specs=pl.BlockSpec((rows, k), lambda i: (i, 0)),
        out_shape=jax.ShapeDtypeStruct((n, k), jnp.float32),
    )(parts, parts)
    return out


# speedup vs baseline: 1.0022x; 1.0022x over previous
import functools

import jax
import jax.numpy as jnp
from jax import lax
from jax.experimental import pallas as pl
from jax.experimental.pallas import tpu as pltpu
from jax.experimental.pallas import tpu_sc as plsc

MIN_NORM = 1e-15
MAXNORM = 1.0 - 4e-3

DP = 48
CHUNK = 128
NW = 32


def _artanh(x):
    x = jnp.clip(x, -1.0 + 1e-7, 1.0 - 1e-7)
    return 0.5 * jnp.log((1.0 + x) / (1.0 - x))


def _rownorm(x):
    return jnp.clip(jnp.sqrt(jnp.sum(x * x, axis=-1, keepdims=True)), MIN_NORM)


def _projf(x):
    norm = _rownorm(x)
    return jnp.where(norm > MAXNORM, x / norm * MAXNORM, x)


def _expmap0f(u):
    un = _rownorm(u)
    return jnp.tanh(un) * u / un


def _logmap0f(p):
    pn = _rownorm(p)
    return _artanh(pn) * p / pn


def _pre_body(x_ref, w_ref, b_ref, out_ref):
    x = x_ref[...]
    w = w_ref[...]
    b = b_ref[...]
    x_norm = _rownorm(x)
    mx = lax.dot_general(x, w, (((1,), (1,)), ((), ())),
                         preferred_element_type=jnp.float32)
    mx_norm = _rownorm(mx)
    res_c = jnp.tanh(mx_norm / x_norm * _artanh(x_norm)) * mx / mx_norm
    cond = jnp.all(mx == 0.0, axis=-1, keepdims=True)
    res = jnp.where(cond, jnp.zeros_like(res_c), res_c)
    res = _projf(res)
    hb = _projf(_expmap0f(b))
    x2 = jnp.sum(res * res, axis=-1, keepdims=True)
    y2 = jnp.sum(hb * hb, axis=-1, keepdims=True)
    xy = jnp.sum(res * hb, axis=-1, keepdims=True)
    num = (1.0 + 2.0 * xy + y2) * res + (1.0 - x2) * hb
    den = jnp.clip(1.0 + 2.0 * xy + x2 * y2, MIN_NORM)
    res = _projf(num / den)
    out_ref[...] = _logmap0f(res)


def _post_body(s0_ref, s1_ref, out_ref):
    s = s0_ref[0] + s1_ref[0]
    h = _projf(_expmap0f(s))
    h = _projf(_expmap0f(_logmap0f(h)))
    out_ref[...] = _logmap0f(h)[:, :out_ref.shape[-1]]


def _sc_agg(xt, src, dst, ew, n_pad, n0, n1):
    rows_per_tile = n_pad // 16
    n_zero_copies = rows_per_tile // CHUNK
    epw0, epw1 = n0 * CHUNK, n1 * CHUNK
    mesh = plsc.VectorSubcoreMesh(core_axis_name="c", subcore_axis_name="s")

    @functools.partial(
        pl.kernel,
        out_type=jax.ShapeDtypeStruct((2, n_pad, DP), jnp.float32),
        mesh=mesh,
        compiler_params=pltpu.CompilerParams(use_tc_tiling_on_sc=False),
        scratch_types=(
            [pltpu.VMEM((CHUNK,), jnp.int32) for _ in range(3)]
            + [pltpu.VMEM((CHUNK,), jnp.int32) for _ in range(3)]
            + [pltpu.VMEM((CHUNK,), jnp.float32) for _ in range(3)]
            + [pltpu.VMEM((CHUNK, DP), jnp.float32) for _ in range(3)]
            + [pltpu.VMEM_SHARED((n_pad, DP), jnp.float32)]
            + [pltpu.SemaphoreType.DMA for _ in range(9)]
        ),
    )
    def body(xt_hbm, src_hbm, dst_hbm, w_hbm, out_hbm,
             s0, s1, s2, d0, d1, d2, w0, w1, w2, r0, r1, r2,
             acc_sh, gs0, gs1, gs2, is0, is1, is2, ss0, ss1, ss2):
        src_v = [s0, s1, s2]
        dst_v = [d0, d1, d2]
        w_v = [w0, w1, w2]
        rows_v = [r0, r1, r2]
        gsem = [gs0, gs1, gs2]
        isem = [is0, is1, is2]
        ssem = [ss0, ss1, ss2]

        cid = lax.axis_index("c")
        sid = lax.axis_index("s")
        base = jnp.where(cid == 0, sid * epw0, 16 * epw0 + sid * epw1)
        nc = jnp.where(cid == 0, n0, n1)
        zero = jnp.zeros((16,), jnp.float32)

        def zrow(i, carry):
            for j in range(DP // 16):
                r0[i, pl.ds(j * 16, 16)] = zero
            return carry
        lax.fori_loop(0, CHUNK, zrow, 0)
        for t in range(n_zero_copies):
            pltpu.sync_copy(
                r0, acc_sh.at[pl.ds(sid * rows_per_tile + t * CHUNK, CHUNK)])
        plsc.subcore_barrier()

        def issue_idx(ci, p):
            off = base + ci * CHUNK
            pltpu.async_copy(src_hbm.at[pl.ds(off, CHUNK)], src_v[p], isem[p])
            pltpu.async_copy(dst_hbm.at[pl.ds(off, CHUNK)], dst_v[p], isem[p])
            pltpu.async_copy(w_hbm.at[pl.ds(off, CHUNK)], w_v[p], isem[p])

        def wait_idx(p):
            pltpu.make_async_copy(
                src_hbm.at[pl.ds(0, CHUNK)], src_v[p], isem[p]).wait()
            pltpu.make_async_copy(
                dst_hbm.at[pl.ds(0, CHUNK)], dst_v[p], isem[p]).wait()
            pltpu.make_async_copy(
                w_hbm.at[pl.ds(0, CHUNK)], w_v[p], isem[p]).wait()

        issue_idx(0, 0)
        issue_idx(1, 1)
        wait_idx(0)
        pltpu.async_copy(xt_hbm.at[src_v[0]], rows_v[0], gsem[0])

        def phase(co, ph):
            p0 = ph % 3
            p1 = (ph + 1) % 3
            p2 = (ph + 2) % 3
            ci = co * 3 + ph
            pltpu.make_async_copy(
                xt_hbm.at[src_v[p0]], rows_v[p0], gsem[p0]).wait()

            wait_idx(p1)
            pltpu.async_copy(xt_hbm.at[src_v[p1]], rows_v[p1], gsem[p1])

            def escale(g, c2):
                wv = w_v[p0][pl.ds(g * 16, 16)]
                for i in range(16):
                    ws = jnp.full((16,), wv[i], jnp.float32)
                    e = g * 16 + i
                    for j in range(DP // 16):
                        rows_v[p0][e, pl.ds(j * 16, 16)] = (
                            rows_v[p0][e, pl.ds(j * 16, 16)] * ws)
                return c2
            lax.fori_loop(0, CHUNK // 16, escale, 0)

            pltpu.async_copy(
                rows_v[p0], acc_sh.at[dst_v[p0]], ssem[p0], add=True)

            @pl.when(ci > 0)
            def _():
                pltpu.make_async_copy(
                    rows_v[p2], acc_sh.at[dst_v[p2]], ssem[p2]).wait()
            issue_idx(ci + 2, p2)

        def cycle(co, carry):
            for ph in range(3):
                phase(co, ph)
            return carry
        lax.fori_loop(0, nc // 3, cycle, 0)

        pltpu.make_async_copy(
            xt_hbm.at[src_v[0]], rows_v[0], gsem[0]).wait()
        wait_idx(1)
        pltpu.make_async_copy(
            rows_v[2], acc_sh.at[dst_v[2]], ssem[2]).wait()

        plsc.subcore_barrier()
        pltpu.sync_copy(acc_sh.at[pl.ds(sid * rows_per_tile, rows_per_tile)],
                        out_hbm.at[cid, pl.ds(sid * rows_per_tile, rows_per_tile)])

    return body(xt, src, dst, ew)


def kernel(x, edge_index, edge_weight, weight, bias):
    n, d = x.shape
    k = weight.shape[0]
    e = edge_index.shape[1]

    wp = jnp.zeros((DP, d), jnp.float32).at[:k].set(weight)
    bp = jnp.zeros((1, DP), jnp.float32).at[0, :k].set(bias)

    rows = 2000
    grid = n // rows
    xt = pl.pallas_call(
        _pre_body,
        grid=(grid,),
        in_specs=[
            pl.BlockSpec((rows, d), lambda i: (i, 0)),
            pl.BlockSpec((DP, d), lambda i: (0, 0)),
            pl.BlockSpec((1, DP), lambda i: (0, 0)),
        ],
        out_specs=pl.BlockSpec((rows, DP), lambda i: (i, 0)),
        out_shape=jax.ShapeDtypeStruct((n, DP), jnp.float32),
    )(x, wp, bp)

    tot = 9 * (-(-e // (16 * CHUNK * 9)))
    n1 = 3 * (-(-(tot * 2) // (7 * 3)))
    n0 = tot - n1
    e_pad = 16 * tot * CHUNK
    e_alloc = e_pad + 2 * CHUNK
    n_pad = 16 * (-(-(n + 1) // (16 * CHUNK)) * CHUNK)
    src = jnp.pad(edge_index[0], (0, e_alloc - e))
    dst = jnp.pad(edge_index[1], (0, e_alloc - e), constant_values=n)
    ew = jnp.pad(edge_weight, (0, e_alloc - e))
    parts = _sc_agg(xt, src, dst, ew, n_pad, n0, n1)

    out = pl.pallas_call(
        _post_body,
        grid=(grid,),
        in_specs=[
            pl.BlockSpec((1, rows, DP), lambda i: (0, i, 0)),
            pl.BlockSpec((1, rows, DP), lambda i: (1, i, 0)),
        ],
        out_specs=pl.BlockSpec((rows, k), lambda i: (i, 0)),
        out_shape=jax.ShapeDtypeStruct((n, k), jnp.float32),
    )(parts, parts)
    return out
